# Initial kernel scaffold; baseline (speedup 1.0000x reference)
import jax
import jax.numpy as jnp
from jax.experimental import pallas as pl


def kernel(x, batch):
    raise NotImplementedError("write your pallas kernel here")


# speedup vs baseline: 2.8767x; 2.8767x over previous
import functools

import jax
import jax.numpy as jnp
from jax import lax
from jax.experimental import pallas as pl
from jax.experimental.pallas import tpu as pltpu
from jax.experimental.pallas import tpu_sc as plsc

N = 100000
D = 128
S = 256
NC = 2
NS = 16
NW = NC * NS
SPW = S // NW
W = 256
G = D // 16
NB = 272


def _sc_body(x_hbm, bnds_hbm, out_hbm, bnds_v, acc_v, buf_v):
    c = lax.axis_index("c")
    s = lax.axis_index("s")
    w = s * NC + c
    seg0 = w * SPW

    pltpu.sync_copy(bnds_hbm, bnds_v)

    zero = jnp.zeros((16,), jnp.float32)
    for si in range(SPW):
        for g in range(G):
            acc_v[si, pl.ds(g * 16, 16)] = zero

    b = [bnds_v[pl.ds(seg0 + si, 16)][0] for si in range(SPW + 1)]
    r_begin = b[0]
    r_end = b[SPW]
    base0 = (r_begin // 8) * 8
    nwin = (r_end - base0 + (W - 1)) // W

    def win_body(k, _):
        win_lo = base0 + k * W
        wstart = jnp.minimum(win_lo, N - W)
        pltpu.sync_copy(x_hbm.at[pl.ds(wstart, W)], buf_v)
        for si in range(SPW):
            b0 = b[si]
            b1 = b[si + 1]
            a = jnp.maximum(b0, win_lo)
            e = jnp.minimum(b1, win_lo + W)
            lo = jnp.clip(a - wstart, 0, W)
            hi = jnp.clip(e - wstart, 0, W)
            hi = jnp.maximum(hi, lo)

            def row_body(j, carry):
                return tuple(carry[g] + buf_v[j, pl.ds(g * 16, 16)]
                             for g in range(G))

            init = tuple(acc_v[si, pl.ds(g * 16, 16)] for g in range(G))
            res = lax.fori_loop(lo, hi, row_body, init)
            for g in range(G):
                acc_v[si, pl.ds(g * 16, 16)] = res[g]
        return 0

    lax.fori_loop(0, nwin, win_body, 0)
    pltpu.sync_copy(acc_v, out_hbm.at[pl.ds(seg0, SPW)])


@functools.partial(
    pl.kernel,
    mesh=plsc.VectorSubcoreMesh(core_axis_name="c", subcore_axis_name="s"),
    out_type=jax.ShapeDtypeStruct((S, D), jnp.float32),
    scratch_types=[
        pltpu.VMEM((NB,), jnp.int32),
        pltpu.VMEM((SPW, D), jnp.float32),
        pltpu.VMEM((W, D), jnp.float32),
    ],
)
def _segment_sum_sc(x_hbm, bnds_hbm, out_hbm, bnds_v, acc_v, buf_v):
    _sc_body(x_hbm, bnds_hbm, out_hbm, bnds_v, acc_v, buf_v)


def kernel(x, batch):
    batch = batch.astype(jnp.int32)
    cuts = jnp.arange(S + 1, dtype=jnp.int32)
    bounds = jnp.searchsorted(batch, cuts).astype(jnp.int32)
    bounds = jnp.concatenate(
        [bounds, jnp.full((NB - (S + 1),), N, dtype=jnp.int32)])
    return _segment_sum_sc(x, bounds)


# speedup vs baseline: 3.3063x; 1.1493x over previous
import functools

import jax
import jax.numpy as jnp
from jax import lax
from jax.experimental import pallas as pl
from jax.experimental.pallas import tpu as pltpu
from jax.experimental.pallas import tpu_sc as plsc

N = 100000
D = 128
S = 256
NC = 2
NS = 16
NW = NC * NS
SPW = S // NW
W = 256
G = D // 16
U = 4
NB = 272


def _sc_body(x_hbm, bnds_hbm, out_hbm, bnds_v, acc_v, buf0_v, buf1_v,
             sem0, sem1):
    c = lax.axis_index("c")
    s = lax.axis_index("s")
    w = s * NC + c
    seg0 = w * SPW

    pltpu.sync_copy(bnds_hbm, bnds_v)

    zero = jnp.zeros((16,), jnp.float32)
    for si in range(SPW):
        for g in range(G):
            acc_v[si, pl.ds(g * 16, 16)] = zero

    b = [bnds_v[pl.ds(seg0 + si, 16)][0] for si in range(SPW + 1)]
    r_begin = b[0]
    r_end = b[SPW]
    base0 = (r_begin // 8) * 8
    nwin = (r_end - base0 + (W - 1)) // W
    npair = (nwin + 1) // 2

    def wstart_of(k):
        return jnp.minimum(base0 + k * W, N - W)

    def start(k, buf, sem):
        @pl.when(k < nwin)
        def _():
            pltpu.async_copy(x_hbm.at[pl.ds(wstart_of(k), W)], buf, sem)

    def wait(k, buf, sem):
        @pl.when(k < nwin)
        def _():
            pltpu.make_async_copy(x_hbm.at[pl.ds(wstart_of(k), W)], buf,
                                  sem).wait()

    def process(k, buf):
        win_lo = base0 + k * W
        wstart = wstart_of(k)
        si_dyn = jnp.int32(0)
        nxt = b[1]
        for si in range(1, SPW):
            inside = b[si] <= win_lo
            si_dyn = jnp.where(inside, jnp.int32(si), si_dyn)
            nxt = jnp.where(inside, b[si + 1], nxt)
        fast = ((win_lo >= r_begin) & (win_lo == wstart)
                & (nxt >= win_lo + W))

        @pl.when(fast)
        def _():
            def rb(t, carry):
                out = list(carry)
                for r in range(U):
                    j = t * U + r
                    for g in range(G):
                        out[g] = out[g] + buf[j, pl.ds(g * 16, 16)]
                return tuple(out)

            res = lax.fori_loop(0, W // U, rb, (zero,) * G)
            for g in range(G):
                sl = pl.ds(g * 16, 16)
                acc_v[si_dyn, sl] = acc_v[si_dyn, sl] + res[g]

        @pl.when(jnp.logical_not(fast))
        def _():
            for si in range(SPW):
                a = jnp.maximum(b[si], win_lo)
                e = jnp.minimum(b[si + 1], win_lo + W)
                lo = jnp.clip(a - wstart, 0, W)
                hi = jnp.clip(e - wstart, 0, W)
                hi = jnp.maximum(hi, lo)

                def row_body(j, carry):
                    return tuple(carry[g] + buf[j, pl.ds(g * 16, 16)]
                                 for g in range(G))

                init = tuple(acc_v[si, pl.ds(g * 16, 16)] for g in range(G))
                res = lax.fori_loop(lo, hi, row_body, init)
                for g in range(G):
                    acc_v[si, pl.ds(g * 16, 16)] = res[g]

    start(jnp.int32(0), buf0_v, sem0)
    start(jnp.int32(1), buf1_v, sem1)

    def pair_body(p, _):
        k0 = 2 * p
        wait(k0, buf0_v, sem0)
        process(k0, buf0_v)
        start(k0 + 2, buf0_v, sem0)
        k1 = 2 * p + 1
        wait(k1, buf1_v, sem1)
        process(k1, buf1_v)
        start(k1 + 2, buf1_v, sem1)
        return 0

    lax.fori_loop(0, npair, pair_body, 0)
    pltpu.sync_copy(acc_v, out_hbm.at[pl.ds(seg0, SPW)])


@functools.partial(
    pl.kernel,
    mesh=plsc.VectorSubcoreMesh(core_axis_name="c", subcore_axis_name="s"),
    out_type=jax.ShapeDtypeStruct((S, D), jnp.float32),
    scratch_types=[
        pltpu.VMEM((NB,), jnp.int32),
        pltpu.VMEM((SPW, D), jnp.float32),
        pltpu.VMEM((W, D), jnp.float32),
        pltpu.VMEM((W, D), jnp.float32),
        pltpu.SemaphoreType.DMA,
        pltpu.SemaphoreType.DMA,
    ],
)
def _segment_sum_sc(x_hbm, bnds_hbm, out_hbm, bnds_v, acc_v, buf0_v, buf1_v,
                    sem0, sem1):
    _sc_body(x_hbm, bnds_hbm, out_hbm, bnds_v, acc_v, buf0_v, buf1_v,
             sem0, sem1)


def kernel(x, batch):
    batch = batch.astype(jnp.int32)
    cuts = jnp.arange(S + 1, dtype=jnp.int32)
    bounds = jnp.searchsorted(batch, cuts).astype(jnp.int32)
    bounds = jnp.concatenate(
        [bounds, jnp.full((NB - (S + 1),), N, dtype=jnp.int32)])
    return _segment_sum_sc(x, bounds)


# speedup vs baseline: 6.1003x; 1.8451x over previous
import functools

import jax
import jax.numpy as jnp
from jax import lax
from jax.experimental import pallas as pl
from jax.experimental.pallas import tpu as pltpu
from jax.experimental.pallas import tpu_sc as plsc

N = 100000
D = 128
S = 256
NC = 2
NS = 16
NW = NC * NS
SPW = S // NW
W = 256
G = D // 16
U = 4
NB = 272


def _sc_body(x_hbm, bnds_hbm, out_hbm, bnds_v, acc_v, buf0_v, buf1_v,
             sem0, sem1):
    c = lax.axis_index("c")
    s = lax.axis_index("s")
    w = s * NC + c
    seg0 = w * SPW

    pltpu.sync_copy(bnds_hbm, bnds_v)

    zero = jnp.zeros((16,), jnp.float32)
    for si in range(SPW):
        for g in range(G):
            acc_v[si, pl.ds(g * 16, 16)] = zero

    b = [bnds_v[pl.ds(seg0 + si, 16)][0] for si in range(SPW + 1)]
    r_begin = b[0]
    r_end = b[SPW]
    base0 = (r_begin // 8) * 8
    nwin = (r_end - base0 + (W - 1)) // W
    npair = (nwin + 1) // 2

    def wstart_of(k):
        return jnp.minimum(base0 + k * W, N - W)

    def start(k, buf, sem):
        @pl.when(k < nwin)
        def _():
            pltpu.async_copy(x_hbm.at[pl.ds(wstart_of(k), W)], buf, sem)

    def wait(k, buf, sem):
        @pl.when(k < nwin)
        def _():
            pltpu.make_async_copy(x_hbm.at[pl.ds(wstart_of(k), W)], buf,
                                  sem).wait()

    def process(k, buf):
        win_lo = base0 + k * W
        wstart = wstart_of(k)
        si_dyn = jnp.int32(0)
        nxt = b[1]
        for si in range(1, SPW):
            inside = b[si] <= win_lo
            si_dyn = jnp.where(inside, jnp.int32(si), si_dyn)
            nxt = jnp.where(inside, b[si + 1], nxt)
        fast = ((win_lo >= r_begin) & (win_lo == wstart)
                & (nxt >= win_lo + W))

        @pl.when(fast)
        def _():
            def rb(t, carry):
                out = list(carry)
                for r in range(U):
                    j = t * U + r
                    for g in range(G):
                        out[g] = out[g] + buf[j, pl.ds(g * 16, 16)]
                return tuple(out)

            res = lax.fori_loop(0, W // U, rb, (zero,) * G)
            for g in range(G):
                sl = pl.ds(g * 16, 16)
                acc_v[si_dyn, sl] = acc_v[si_dyn, sl] + res[g]

        @pl.when(jnp.logical_not(fast))
        def _():
            for si in range(SPW):
                a = jnp.maximum(b[si], win_lo)
                e = jnp.minimum(b[si + 1], win_lo + W)
                lo = jnp.clip(a - wstart, 0, W)
                hi = jnp.clip(e - wstart, 0, W)
                hi = jnp.maximum(hi, lo)

                def row_body(j, carry):
                    return tuple(carry[g] + buf[j, pl.ds(g * 16, 16)]
                                 for g in range(G))

                init = tuple(acc_v[si, pl.ds(g * 16, 16)] for g in range(G))
                res = lax.fori_loop(lo, hi, row_body, init)
                for g in range(G):
                    acc_v[si, pl.ds(g * 16, 16)] = res[g]

    start(jnp.int32(0), buf0_v, sem0)
    start(jnp.int32(1), buf1_v, sem1)

    def pair_body(p, _):
        k0 = 2 * p
        wait(k0, buf0_v, sem0)
        process(k0, buf0_v)
        start(k0 + 2, buf0_v, sem0)
        k1 = 2 * p + 1
        wait(k1, buf1_v, sem1)
        process(k1, buf1_v)
        start(k1 + 2, buf1_v, sem1)
        return 0

    lax.fori_loop(0, npair, pair_body, 0)
    pltpu.sync_copy(acc_v, out_hbm.at[pl.ds(seg0, SPW)])


@functools.partial(
    pl.kernel,
    mesh=plsc.VectorSubcoreMesh(core_axis_name="c", subcore_axis_name="s"),
    out_type=jax.ShapeDtypeStruct((S, D), jnp.float32),
    scratch_types=[
        pltpu.VMEM((NB,), jnp.int32),
        pltpu.VMEM((SPW, D), jnp.float32),
        pltpu.VMEM((W, D), jnp.float32),
        pltpu.VMEM((W, D), jnp.float32),
        pltpu.SemaphoreType.DMA,
        pltpu.SemaphoreType.DMA,
    ],
)
def _segment_sum_sc(x_hbm, bnds_hbm, out_hbm, bnds_v, acc_v, buf0_v, buf1_v,
                    sem0, sem1):
    _sc_body(x_hbm, bnds_hbm, out_hbm, bnds_v, acc_v, buf0_v, buf1_v,
             sem0, sem1)


def kernel(x, batch):
    batch = batch.astype(jnp.int32)
    cuts = jnp.arange(S + 1, dtype=jnp.int32)
    bounds = jnp.searchsorted(batch, cuts,
                              method="compare_all").astype(jnp.int32)
    bounds = jnp.concatenate(
        [bounds, jnp.full((NB - (S + 1),), N, dtype=jnp.int32)])
    return _segment_sum_sc(x, bounds)
